# Initial kernel scaffold; baseline (speedup 1.0000x reference)
import jax
import jax.numpy as jnp
from jax.experimental import pallas as pl


def kernel(input_data, input_coords, input_coords_inv, W_red, b_red, W_fcl0, b_fcl0, W_fcl1, b_fcl1, W_fcl2, b_fcl2, W_fcl3, b_fcl3, W_fcs0, b_fcs0, W_fcs1, b_fcs1, W_fcs2, b_fcs2, W_fcs3, b_fcs3, W_fc, W_out, W_lo1, W_lo2, b_lo2):
    raise NotImplementedError("write your pallas kernel here")


# speedup vs baseline: 1.0607x; 1.0607x over previous
import functools

import jax
import jax.numpy as jnp
from jax.experimental import pallas as pl

N_BLK = 2000


def _phase_a_body(x_ref, wred_ref, bred_ref, wcat_ref, rf_ref, g_ref):
    x = x_ref[...]
    rf = jnp.maximum(
        jnp.dot(x, wred_ref[...], preferred_element_type=jnp.float32)
        + bred_ref[...], 0.0)
    rf_ref[...] = rf
    g_ref[...] = jnp.dot(rf, wcat_ref[...], preferred_element_type=jnp.float32)


def _phase_b_body(m_ref, rf_ref, bfcl_ref, wfc_ref, wfcs_ref, bfcs_ref,
                  wout_ref, wlo1_ref, wlo2_ref, blo2_ref, out_ref):
    att = jnp.maximum(m_ref[...] + bfcl_ref[...], 0.0)
    a0, a1, a2, a3 = att[:, 0:64], att[:, 64:128], att[:, 128:192], att[:, 192:256]
    feat_s = a0 + a1 + a2 + a3
    feat_z = jnp.maximum(
        jnp.dot(feat_s, wfc_ref[...], preferred_element_type=jnp.float32), 0.0)
    av = jax.nn.sigmoid(
        jnp.dot(feat_z, wfcs_ref[...], preferred_element_type=jnp.float32)
        + bfcs_ref[...])
    sf = (a0 * av[:, 0:64] + a1 * av[:, 64:128]
          + a2 * av[:, 128:192] + a3 * av[:, 192:256])
    out128 = jnp.dot(sf, wout_ref[...], preferred_element_type=jnp.float32)
    rf = rf_ref[...]
    h = jnp.maximum(
        jnp.dot(rf, wlo1_ref[:128, :], preferred_element_type=jnp.float32)
        + jnp.dot(out128, wlo1_ref[128:, :], preferred_element_type=jnp.float32),
        0.0)
    out_ref[...] = (jnp.dot(h, wlo2_ref[...], preferred_element_type=jnp.float32)
                    + blo2_ref[...])


def _rows(blk):
    return pl.BlockSpec((blk, None), lambda i: (i, 0))


def _full(*shape):
    return pl.BlockSpec(shape, lambda i: tuple(0 for _ in shape))


def _phase_a(x, w_red, b_red, w_cat):
    n, cin = x.shape
    cr = w_red.shape[1]
    cg = w_cat.shape[1]
    return pl.pallas_call(
        _phase_a_body,
        grid=(n // N_BLK,),
        in_specs=[
            pl.BlockSpec((N_BLK, cin), lambda i: (i, 0)),
            _full(cin, cr),
            _full(1, cr),
            _full(cr, cg),
        ],
        out_specs=[
            pl.BlockSpec((N_BLK, cr), lambda i: (i, 0)),
            pl.BlockSpec((N_BLK, cg), lambda i: (i, 0)),
        ],
        out_shape=[
            jax.ShapeDtypeStruct((n, cr), jnp.float32),
            jax.ShapeDtypeStruct((n, cg), jnp.float32),
        ],
    )(x, w_red, b_red.reshape(1, cr), w_cat)


def _phase_b(m_cat, rf, b_fcl_cat, w_fc, w_fcs_cat, b_fcs_cat, w_out,
             w_lo1, w_lo2, b_lo2):
    n = m_cat.shape[0]
    cout = w_lo2.shape[1]
    return pl.pallas_call(
        _phase_b_body,
        grid=(n // N_BLK,),
        in_specs=[
            pl.BlockSpec((N_BLK, 256), lambda i: (i, 0)),
            pl.BlockSpec((N_BLK, 128), lambda i: (i, 0)),
            _full(1, 256),
            _full(64, 64),
            _full(64, 256),
            _full(1, 256),
            _full(64, 128),
            _full(256, 128),
            _full(128, 256),
            _full(1, 256),
        ],
        out_specs=pl.BlockSpec((N_BLK, cout), lambda i: (i, 0)),
        out_shape=jax.ShapeDtypeStruct((n, cout), jnp.float32),
    )(m_cat, rf, b_fcl_cat.reshape(1, 256), w_fc, w_fcs_cat,
      b_fcs_cat.reshape(1, 256), w_out, w_lo1, w_lo2, b_lo2.reshape(1, -1))


def kernel(input_data, input_coords, input_coords_inv, W_red, b_red,
           W_fcl0, b_fcl0, W_fcl1, b_fcl1, W_fcl2, b_fcl2, W_fcl3, b_fcl3,
           W_fcs0, b_fcs0, W_fcs1, b_fcs1, W_fcs2, b_fcs2, W_fcs3, b_fcs3,
           W_fc, W_out, W_lo1, W_lo2, b_lo2):
    n = input_data.shape[0]
    w_cat = jnp.concatenate([W_fcl0, W_fcl1, W_fcl2, W_fcl3], axis=1)
    rf, g = _phase_a(input_data, W_red, b_red, w_cat)

    means = []
    ones = jnp.ones((n, 1), dtype=jnp.float32)
    for j, ps in enumerate([2, 4, 6, 8]):
        index = jnp.concatenate(
            [input_coords[:, :1], input_coords[:, 1:] // ps], axis=1)
        _, unq_inv = jnp.unique(index, axis=0, return_inverse=True,
                                size=n, fill_value=0)
        unq_inv = unq_inv.ravel()
        gj = g[:, 64 * j:64 * (j + 1)]
        sums = jax.ops.segment_sum(gj, unq_inv, num_segments=n)
        cnt = jax.ops.segment_sum(ones, unq_inv, num_segments=n)
        fkm = sums / jnp.maximum(cnt, 1.0)
        means.append(fkm[unq_inv])
    m_cat = jnp.concatenate(means, axis=1)

    b_fcl_cat = jnp.concatenate([b_fcl0, b_fcl1, b_fcl2, b_fcl3])
    w_fcs_cat = jnp.concatenate([W_fcs0, W_fcs1, W_fcs2, W_fcs3], axis=1)
    b_fcs_cat = jnp.concatenate([b_fcs0, b_fcs1, b_fcs2, b_fcs3])
    proj = _phase_b(m_cat, rf, b_fcl_cat, W_fc, w_fcs_cat, b_fcs_cat,
                    W_out, W_lo1, W_lo2, b_lo2)
    return proj[input_coords_inv]
